# Initial kernel scaffold; baseline (speedup 1.0000x reference)
import jax
import jax.numpy as jnp
from jax.experimental import pallas as pl


def kernel(x, y, mask, W1, b1, W2, b2, W3, b3):
    raise NotImplementedError("write your pallas kernel here")


# speedup vs baseline: 3.3908x; 3.3908x over previous
import jax
import jax.numpy as jnp
from jax.experimental import pallas as pl
from jax.experimental.pallas import tpu as pltpu

B, N = 16, 4096
X_DIM, Y_DIM, H_DIM, R_DIM = 16, 16, 64, 64


def _body(x_ref, y_ref, m_ref, w1_ref, b1_ref, w2_ref, b2_ref, w3_ref, b3_ref,
          out_ref):
    xb = x_ref[0]
    yb = y_ref[0]
    m = m_ref[0]
    nn_in = jnp.concatenate([xb, yb], axis=1)
    h = jnp.dot(nn_in, w1_ref[...], preferred_element_type=jnp.float32)
    h = jnp.maximum(h + b1_ref[...], 0.0)
    h2 = jnp.dot(h, w2_ref[...], preferred_element_type=jnp.float32)
    h2 = jnp.maximum(h2 + b2_ref[...], 0.0)
    s = jnp.sum(h2 * m, axis=0, keepdims=True)
    cnt = jnp.sum(m)
    r = jnp.dot(s, w3_ref[...], preferred_element_type=jnp.float32)
    r = r + cnt * b3_ref[...]
    out_ref[0] = r / jnp.maximum(cnt, 1.0)


def kernel(x, y, mask, W1, b1, W2, b2, W3, b3):
    mf = mask.astype(jnp.float32).reshape(B, N, 1)
    b1r = b1.reshape(1, H_DIM)
    b2r = b2.reshape(1, H_DIM)
    b3r = b3.reshape(1, R_DIM)

    out = pl.pallas_call(
        _body,
        grid=(B,),
        in_specs=[
            pl.BlockSpec((1, N, X_DIM), lambda b: (b, 0, 0)),
            pl.BlockSpec((1, N, Y_DIM), lambda b: (b, 0, 0)),
            pl.BlockSpec((1, N, 1), lambda b: (b, 0, 0)),
            pl.BlockSpec((X_DIM + Y_DIM, H_DIM), lambda b: (0, 0)),
            pl.BlockSpec((1, H_DIM), lambda b: (0, 0)),
            pl.BlockSpec((H_DIM, H_DIM), lambda b: (0, 0)),
            pl.BlockSpec((1, H_DIM), lambda b: (0, 0)),
            pl.BlockSpec((H_DIM, R_DIM), lambda b: (0, 0)),
            pl.BlockSpec((1, R_DIM), lambda b: (0, 0)),
        ],
        out_specs=pl.BlockSpec((1, 1, R_DIM), lambda b: (b, 0, 0)),
        out_shape=jax.ShapeDtypeStruct((B, 1, R_DIM), jnp.float32),
        compiler_params=pltpu.CompilerParams(
            dimension_semantics=("arbitrary",),
        ),
    )(x, y, mf, W1, b1r, W2, b2r, W3, b3r)
    return out.reshape(B, R_DIM)


# speedup vs baseline: 3.3946x; 1.0011x over previous
import jax
import jax.numpy as jnp
from jax.experimental import pallas as pl
from jax.experimental.pallas import tpu as pltpu

B, N = 16, 4096
X_DIM, Y_DIM, H_DIM, R_DIM = 16, 16, 64, 64


def _body(x_ref, y_ref, m_ref, w1_ref, b1_ref, w2_ref, b2_ref, w3_ref, b3_ref,
          out_ref):
    xb = x_ref[0]
    yb = y_ref[0]
    m = m_ref[0]
    nn_in = jnp.concatenate([xb, yb], axis=1)
    h = jnp.dot(nn_in, w1_ref[...], preferred_element_type=jnp.float32)
    h = jnp.maximum(h + b1_ref[...], 0.0)
    h2 = jnp.dot(h, w2_ref[...], preferred_element_type=jnp.float32)
    h2 = jnp.maximum(h2 + b2_ref[...], 0.0)
    s = jnp.sum(h2 * m, axis=0, keepdims=True)
    cnt = jnp.sum(m)
    r = jnp.dot(s, w3_ref[...], preferred_element_type=jnp.float32)
    r = r + cnt * b3_ref[...]
    out_ref[0] = r / jnp.maximum(cnt, 1.0)


def kernel(x, y, mask, W1, b1, W2, b2, W3, b3):
    mf = mask.astype(jnp.float32).reshape(B, N, 1)
    b1r = b1.reshape(1, H_DIM)
    b2r = b2.reshape(1, H_DIM)
    b3r = b3.reshape(1, R_DIM)

    out = pl.pallas_call(
        _body,
        grid=(B,),
        in_specs=[
            pl.BlockSpec((1, N, X_DIM), lambda b: (b, 0, 0)),
            pl.BlockSpec((1, N, Y_DIM), lambda b: (b, 0, 0)),
            pl.BlockSpec((1, N, 1), lambda b: (b, 0, 0)),
            pl.BlockSpec((X_DIM + Y_DIM, H_DIM), lambda b: (0, 0)),
            pl.BlockSpec((1, H_DIM), lambda b: (0, 0)),
            pl.BlockSpec((H_DIM, H_DIM), lambda b: (0, 0)),
            pl.BlockSpec((1, H_DIM), lambda b: (0, 0)),
            pl.BlockSpec((H_DIM, R_DIM), lambda b: (0, 0)),
            pl.BlockSpec((1, R_DIM), lambda b: (0, 0)),
        ],
        out_specs=pl.BlockSpec((1, 1, R_DIM), lambda b: (b, 0, 0)),
        out_shape=jax.ShapeDtypeStruct((B, 1, R_DIM), jnp.float32),
        compiler_params=pltpu.CompilerParams(
            dimension_semantics=("parallel",),
        ),
    )(x, y, mf, W1, b1r, W2, b2r, W3, b3r)
    return out.reshape(B, R_DIM)
